# baseline (speedup 1.0000x reference)
import functools

import jax
import jax.numpy as jnp
from jax import lax
from jax.experimental import pallas as pl
from jax.experimental.pallas import tpu as pltpu
from jax.experimental.pallas import tpu_sc as plsc

N1, N2 = 4096, 1024
D = 128
DA = 144
E1, E2 = 320000, 131072
NC, NS = 2, 16
NW = NC * NS
CHUNK = 128

NCH1 = 80
E1P = NW * NCH1 * CHUNK
T1_ROWS = 4096
ACC1_ROWS = 1152

NCH2 = 32
T2_ROWS = 1024
ACC2_ROWS = 1024


def _make_seg_sum(nch, table_rows, acc_rows, filter_below=None):
    rpt = acc_rows // NS
    mesh = plsc.VectorSubcoreMesh(core_axis_name="c", subcore_axis_name="s")
    ncap = nch * CHUNK + 2 * CHUNK
    filt = filter_below is not None

    nchs = nch + 2 if filt else nch
    scratch = [
        pltpu.VMEM((nchs, CHUNK), jnp.int32),
        pltpu.VMEM((nchs, CHUNK), jnp.int32),
        pltpu.VMEM((CHUNK, DA), jnp.float32),
        pltpu.VMEM((CHUNK, DA), jnp.float32),
        pltpu.VMEM_SHARED((acc_rows, DA), jnp.float32),
        pltpu.SemaphoreType.DMA,
        pltpu.SemaphoreType.DMA,
    ]

    @functools.partial(
        pl.kernel,
        mesh=mesh,
        compiler_params=pltpu.CompilerParams(
            use_tc_tiling_on_sc=False, needs_layout_passes=False),
        out_type=jax.ShapeDtypeStruct((NC, acc_rows, DA), jnp.float32),
        scratch_types=scratch,
    )
    def seg_kernel(table_hbm, src_hbm, dst_hbm, out_hbm,
                   src_v, dst_v, rows0, rows1, acc, sem, sem_s):
        cid = lax.axis_index("c")
        sid = lax.axis_index("s")
        wid = sid * NC + cid
        i32 = jnp.int32

        def _zrow(r, carry):
            for j in range(DA // 16):
                rows0[r, pl.ds(j * 16, 16)] = jnp.zeros((16,), jnp.float32)
            return carry
        lax.fori_loop(0, CHUNK, _zrow, 0)
        r0 = sid * rpt
        off = 0
        while off < rpt:
            n = min(CHUNK, rpt - off)
            pltpu.sync_copy(rows0.at[pl.ds(0, n)], acc.at[pl.ds(r0 + off, n)])
            off += n
        plsc.subcore_barrier()

        c0 = wid * nch
        pltpu.sync_copy(src_hbm.at[pl.ds(c0, nch)], src_v.at[pl.ds(0, nch)])
        pltpu.sync_copy(dst_hbm.at[pl.ds(c0, nch)], dst_v.at[pl.ds(0, nch)])

        if filt:
            lanes16 = lax.iota(i32, 16)
            trash = (nchs * CHUNK - 16) + lanes16

            def _scan_chunk(c, off_vec):
                for j in range(CHUNK // 16):
                    sv = src_v[c, pl.ds(j * 16, 16)]
                    dv = dst_v[c, pl.ds(j * 16, 16)]
                    m = dv < filter_below
                    mi = m.astype(i32)
                    cnt = plsc.all_reduce_population_count(m)
                    pos = plsc.cumsum(mi) + (off_vec - 1)
                    idx = jnp.where(m, pos, trash)
                    row = jnp.right_shift(idx, 7)
                    col = jnp.bitwise_and(idx, CHUNK - 1)
                    plsc.store_scatter(src_v, [row, col], sv)
                    plsc.store_scatter(dst_v, [row, col], dv)
                    off_vec = off_vec + cnt
                return off_vec
            nedge = lax.fori_loop(
                0, nch, _scan_chunk, jnp.zeros((16,), i32))[0]

            for k in range(2 * CHUNK // 16):
                pp = nedge + 16 * k + lanes16
                prow = jnp.right_shift(pp, 7)
                pcol = jnp.bitwise_and(pp, CHUNK - 1)
                plsc.store_scatter(
                    src_v, [prow, pcol], 16 * k + lanes16)
                plsc.store_scatter(
                    dst_v, [prow, pcol],
                    filter_below + ((16 * k + lanes16) % CHUNK))
            nch_d = jnp.maximum(2 * ((nedge + 2 * CHUNK - 1) // (2 * CHUNK)),
                                2)
        else:
            nch_d = nch

        def _gidx(g):
            return src_v.at[g]

        def _didx(g):
            return dst_v.at[g]

        pltpu.async_copy(table_hbm.at[_gidx(0)], rows0, sem)

        def _swait():
            pltpu.make_async_copy(rows0, acc.at[_didx(0)], sem_s).wait()

        def _pair(h, carry):
            g0 = 2 * h
            g1 = g0 + 1
            pltpu.make_async_copy(table_hbm.at[_gidx(g0)], rows0, sem).wait()

            @pl.when(h > 0)
            def _drain_prev():
                _swait()

            pltpu.async_copy(table_hbm.at[_gidx(g1)], rows1, sem)
            pltpu.async_copy(rows0, acc.at[_didx(g0)], sem_s, add=True)
            pltpu.make_async_copy(table_hbm.at[_gidx(g1)], rows1, sem).wait()
            _swait()

            @pl.when(g1 + 1 < nch_d)
            def _prefetch():
                pltpu.async_copy(table_hbm.at[_gidx(g1 + 1)], rows0, sem)

            pltpu.async_copy(rows1, acc.at[_didx(g1)], sem_s, add=True)
            return carry
        lax.fori_loop(0, nch_d // 2, _pair, 0)
        _swait()
        plsc.subcore_barrier()

        off = 0
        while off < rpt:
            n = min(CHUNK, rpt - off)
            pltpu.sync_copy(acc.at[pl.ds(r0 + off, n)],
                            out_hbm.at[cid, pl.ds(r0 + off, n)])
            off += n

    return seg_kernel


_seg1 = _make_seg_sum(NCH1, T1_ROWS, ACC1_ROWS, filter_below=N2)
_seg2 = _make_seg_sum(NCH2, T2_ROWS, ACC2_ROWS)


def _tc1_body(p_ref, xd_ref, wl_ref, b_ref, wr_ref, o_ref):
    s = p_ref[0, :N2] + p_ref[1, :N2]
    cnt = s[:, D:D + 1]
    mean = s[:, :D] / jnp.maximum(cnt, 1.0)
    h = mean @ wl_ref[...] + b_ref[...] + xd_ref[...] @ wr_ref[...]
    h = jnp.maximum(h, 0.0)
    o_ref[...] = jnp.concatenate(
        [h, jnp.ones((N2, 1), jnp.float32), jnp.zeros((N2, DA - D - 1),
                                                      jnp.float32)], axis=1)


def _tc2_body(p_ref, h_ref, wl_ref, b_ref, wr_ref, o_ref):
    s = p_ref[0] + p_ref[1]
    mean = s[:, :D] / jnp.maximum(s[:, D:D + 1], 1.0)
    z = mean @ wl_ref[...] + b_ref[...] + h_ref[...][:, :D] @ wr_ref[...]
    m = jnp.max(z, axis=1, keepdims=True)
    lse = jnp.log(jnp.sum(jnp.exp(z - m), axis=1, keepdims=True)) + m
    o_ref[...] = z - lse


_tc1 = pl.pallas_call(
    _tc1_body, out_shape=jax.ShapeDtypeStruct((N2, DA), jnp.float32))
_tc2 = pl.pallas_call(
    _tc2_body, out_shape=jax.ShapeDtypeStruct((N2, D), jnp.float32))


def kernel(x, edge_index1, edge_index2, W1_l, b1_l, W1_r, W2_l, b2_l, W2_r):
    f32 = jnp.float32
    i32 = jnp.int32
    src1 = edge_index1[0].astype(i32)
    dst1 = edge_index1[1].astype(i32)
    src2 = edge_index2[0].astype(i32)
    dst2 = edge_index2[1].astype(i32)

    pad1 = E1P - E1
    src1p = jnp.concatenate(
        [src1, jnp.zeros((pad1,), i32)]).reshape(-1, CHUNK)
    dst1p = jnp.concatenate(
        [dst1, jnp.full((pad1,), N2, i32)]).reshape(-1, CHUNK)
    src2r = src2.reshape(-1, CHUNK)
    dst2r = dst2.reshape(-1, CHUNK)

    xa = jnp.concatenate(
        [x[:N1], jnp.ones((N1, 1), f32), jnp.zeros((N1, DA - D - 1), f32)],
        axis=1)

    part1 = _seg1(xa, src1p, dst1p)
    ha = _tc1(part1, x[:N2], W1_l, b1_l.reshape(1, D), W1_r)

    part2 = _seg2(ha, src2r, dst2r)
    out = _tc2(part2, ha, W2_l, b2_l.reshape(1, D), W2_r)
    return out


# speedup vs baseline: 1.0539x; 1.0539x over previous
import functools

import jax
import jax.numpy as jnp
from jax import lax
from jax.experimental import pallas as pl
from jax.experimental.pallas import tpu as pltpu
from jax.experimental.pallas import tpu_sc as plsc

N1, N2 = 4096, 1024
D = 128
DA = 144
E1, E2 = 320000, 131072
NC, NS = 2, 16
NW = NC * NS
CHUNK = 128

NCH1 = 80
E1P = NW * NCH1 * CHUNK
T1_ROWS = 4096
ACC1_ROWS = 1152

NCH2 = 32
T2_ROWS = 1024
ACC2_ROWS = 1024


def _make_seg_sum(nch, table_rows, acc_rows, filter_below=None):
    rpt = acc_rows // NS
    mesh = plsc.VectorSubcoreMesh(core_axis_name="c", subcore_axis_name="s")
    ncap = nch * CHUNK + 2 * CHUNK
    filt = filter_below is not None

    nchs = nch + 2 if filt else nch
    scratch = [
        pltpu.VMEM((nchs, CHUNK), jnp.int32),
        pltpu.VMEM((nchs, CHUNK), jnp.int32),
        pltpu.VMEM((CHUNK, DA), jnp.float32),
        pltpu.VMEM((CHUNK, DA), jnp.float32),
        pltpu.VMEM_SHARED((acc_rows, DA), jnp.float32),
        pltpu.SemaphoreType.DMA,
        pltpu.SemaphoreType.DMA,
    ]

    @functools.partial(
        pl.kernel,
        mesh=mesh,
        compiler_params=pltpu.CompilerParams(
            use_tc_tiling_on_sc=False, needs_layout_passes=False),
        out_type=jax.ShapeDtypeStruct((NC, acc_rows, DA), jnp.float32),
        scratch_types=scratch,
    )
    def seg_kernel(table_hbm, edge_hbm, out_hbm,
                   src_v, dst_v, rows0, rows1, acc, sem, sem_s):
        cid = lax.axis_index("c")
        sid = lax.axis_index("s")
        wid = sid * NC + cid
        i32 = jnp.int32

        def _zrow(r, carry):
            for j in range(DA // 16):
                rows0[r, pl.ds(j * 16, 16)] = jnp.zeros((16,), jnp.float32)
            return carry
        lax.fori_loop(0, CHUNK, _zrow, 0)
        r0 = sid * rpt
        off = 0
        while off < rpt:
            n = min(CHUNK, rpt - off)
            pltpu.sync_copy(rows0.at[pl.ds(0, n)], acc.at[pl.ds(r0 + off, n)])
            off += n
        plsc.subcore_barrier()

        c0 = wid * nch
        pltpu.sync_copy(edge_hbm.at[0, pl.ds(c0, nch)], src_v.at[pl.ds(0, nch)])
        pltpu.sync_copy(edge_hbm.at[1, pl.ds(c0, nch)], dst_v.at[pl.ds(0, nch)])

        if filt:
            lanes16 = lax.iota(i32, 16)
            trash = (nchs * CHUNK - 16) + lanes16

            def _scan_chunk(c, off_vec):
                for j in range(CHUNK // 16):
                    sv = src_v[c, pl.ds(j * 16, 16)]
                    dv = dst_v[c, pl.ds(j * 16, 16)]
                    m = dv < filter_below
                    mi = m.astype(i32)
                    cnt = plsc.all_reduce_population_count(m)
                    pos = plsc.cumsum(mi) + (off_vec - 1)
                    idx = jnp.where(m, pos, trash)
                    row = jnp.right_shift(idx, 7)
                    col = jnp.bitwise_and(idx, CHUNK - 1)
                    plsc.store_scatter(src_v, [row, col], sv)
                    plsc.store_scatter(dst_v, [row, col], dv)
                    off_vec = off_vec + cnt
                return off_vec
            nedge = lax.fori_loop(
                0, nch, _scan_chunk, jnp.zeros((16,), i32))[0]

            for k in range(2 * CHUNK // 16):
                pp = nedge + 16 * k + lanes16
                prow = jnp.right_shift(pp, 7)
                pcol = jnp.bitwise_and(pp, CHUNK - 1)
                plsc.store_scatter(
                    src_v, [prow, pcol], 16 * k + lanes16)
                plsc.store_scatter(
                    dst_v, [prow, pcol],
                    filter_below + ((16 * k + lanes16) % CHUNK))
            nch_d = jnp.maximum(2 * ((nedge + 2 * CHUNK - 1) // (2 * CHUNK)),
                                2)
        else:
            nch_d = nch

        def _gidx(g):
            return src_v.at[g]

        def _didx(g):
            return dst_v.at[g]

        pltpu.async_copy(table_hbm.at[_gidx(0)], rows0, sem)

        def _swait():
            pltpu.make_async_copy(rows0, acc.at[_didx(0)], sem_s).wait()

        def _pair(h, carry):
            g0 = 2 * h
            g1 = g0 + 1
            pltpu.make_async_copy(table_hbm.at[_gidx(g0)], rows0, sem).wait()

            @pl.when(h > 0)
            def _drain_prev():
                _swait()

            pltpu.async_copy(table_hbm.at[_gidx(g1)], rows1, sem)
            pltpu.async_copy(rows0, acc.at[_didx(g0)], sem_s, add=True)
            pltpu.make_async_copy(table_hbm.at[_gidx(g1)], rows1, sem).wait()
            _swait()

            @pl.when(g1 + 1 < nch_d)
            def _prefetch():
                pltpu.async_copy(table_hbm.at[_gidx(g1 + 1)], rows0, sem)

            pltpu.async_copy(rows1, acc.at[_didx(g1)], sem_s, add=True)
            return carry
        lax.fori_loop(0, nch_d // 2, _pair, 0)
        _swait()
        plsc.subcore_barrier()

        off = 0
        while off < rpt:
            n = min(CHUNK, rpt - off)
            pltpu.sync_copy(acc.at[pl.ds(r0 + off, n)],
                            out_hbm.at[cid, pl.ds(r0 + off, n)])
            off += n

    return seg_kernel


_seg1 = _make_seg_sum(NCH1, T1_ROWS, ACC1_ROWS, filter_below=N2)
_seg2 = _make_seg_sum(NCH2, T2_ROWS, ACC2_ROWS)


def _tc1_body(p_ref, xd_ref, wl_ref, b_ref, wr_ref, o_ref):
    s = p_ref[0, :N2] + p_ref[1, :N2]
    cnt = s[:, D:D + 1]
    mean = s[:, :D] / jnp.maximum(cnt, 1.0)
    h = mean @ wl_ref[...] + b_ref[...] + xd_ref[...] @ wr_ref[...]
    h = jnp.maximum(h, 0.0)
    o_ref[...] = jnp.concatenate(
        [h, jnp.ones((N2, 1), jnp.float32), jnp.zeros((N2, DA - D - 1),
                                                      jnp.float32)], axis=1)


def _tc2_body(p_ref, h_ref, wl_ref, b_ref, wr_ref, o_ref):
    s = p_ref[0] + p_ref[1]
    mean = s[:, :D] / jnp.maximum(s[:, D:D + 1], 1.0)
    z = mean @ wl_ref[...] + b_ref[...] + h_ref[...][:, :D] @ wr_ref[...]
    m = jnp.max(z, axis=1, keepdims=True)
    lse = jnp.log(jnp.sum(jnp.exp(z - m), axis=1, keepdims=True)) + m
    o_ref[...] = z - lse


_tc1 = pl.pallas_call(
    _tc1_body, out_shape=jax.ShapeDtypeStruct((N2, DA), jnp.float32))
_tc2 = pl.pallas_call(
    _tc2_body, out_shape=jax.ShapeDtypeStruct((N2, D), jnp.float32))


def kernel(x, edge_index1, edge_index2, W1_l, b1_l, W1_r, W2_l, b2_l, W2_r):
    f32 = jnp.float32
    i32 = jnp.int32
    pad1 = E1P - E1
    pads = jnp.concatenate(
        [jnp.zeros((1, pad1), i32), jnp.full((1, pad1), N2, i32)], axis=0)
    e1 = jnp.concatenate(
        [edge_index1.astype(i32), pads], axis=1).reshape(2, -1, CHUNK)
    e2 = edge_index2.astype(i32).reshape(2, -1, CHUNK)

    xa = jnp.concatenate(
        [x[:N1], jnp.ones((N1, 1), f32), jnp.zeros((N1, DA - D - 1), f32)],
        axis=1)

    part1 = _seg1(xa, e1)
    ha = _tc1(part1, x[:N2], W1_l, b1_l.reshape(1, D), W1_r)

    part2 = _seg2(ha, e2)
    out = _tc2(part2, ha, W2_l, b2_l.reshape(1, D), W2_r)
    return out


# speedup vs baseline: 1.0543x; 1.0004x over previous
import functools

import jax
import jax.numpy as jnp
from jax import lax
from jax.experimental import pallas as pl
from jax.experimental.pallas import tpu as pltpu
from jax.experimental.pallas import tpu_sc as plsc

N1, N2 = 4096, 1024
D = 128
DA = 144
E1, E2 = 320000, 131072
NC, NS = 2, 16
NW = NC * NS
CHUNK = 128

NCH1 = 80
E1P = NW * NCH1 * CHUNK
T1_ROWS = 4096
ACC1_ROWS = 1152

NCH2 = 32
T2_ROWS = 1024
ACC2_ROWS = 1024


def _make_seg_sum(nch, table_rows, acc_rows, filter_below=None):
    rpt = acc_rows // NS
    mesh = plsc.VectorSubcoreMesh(core_axis_name="c", subcore_axis_name="s")
    filt = filter_below is not None

    nchs = nch + 2 if filt else nch
    scratch = [
        pltpu.VMEM((nchs, CHUNK), jnp.int32),
        pltpu.VMEM((nchs, CHUNK), jnp.int32),
        pltpu.VMEM((CHUNK, DA), jnp.float32),
        pltpu.VMEM((CHUNK, DA), jnp.float32),
        pltpu.VMEM_SHARED((acc_rows, DA), jnp.float32),
        pltpu.SemaphoreType.DMA,
        pltpu.SemaphoreType.DMA,
    ]

    @functools.partial(
        pl.kernel,
        mesh=mesh,
        compiler_params=pltpu.CompilerParams(
            use_tc_tiling_on_sc=False, needs_layout_passes=False),
        out_type=jax.ShapeDtypeStruct((NC, acc_rows, DA), jnp.float32),
        scratch_types=scratch,
    )
    def seg_kernel(table_hbm, edge_hbm, out_hbm,
                   src_v, dst_v, rows0, rows1, acc, sem, sem_s):
        cid = lax.axis_index("c")
        sid = lax.axis_index("s")
        wid = sid * NC + cid
        i32 = jnp.int32

        def _zrow(r, carry):
            for j in range(DA // 16):
                rows0[r, pl.ds(j * 16, 16)] = jnp.zeros((16,), jnp.float32)
            return carry
        lax.fori_loop(0, CHUNK, _zrow, 0)
        r0 = sid * rpt
        off = 0
        while off < rpt:
            n = min(CHUNK, rpt - off)
            pltpu.sync_copy(rows0.at[pl.ds(0, n)], acc.at[pl.ds(r0 + off, n)])
            off += n
        plsc.subcore_barrier()

        c0 = wid * nch
        pltpu.sync_copy(edge_hbm.at[0, pl.ds(c0, nch)], src_v.at[pl.ds(0, nch)])
        pltpu.sync_copy(edge_hbm.at[1, pl.ds(c0, nch)], dst_v.at[pl.ds(0, nch)])

        if filt:
            lanes16 = lax.iota(i32, 16)
            trash = (nchs * CHUNK - 16) + lanes16

            def _scan_chunk(c, off_vec):
                for j in range(CHUNK // 16):
                    sv = src_v[c, pl.ds(j * 16, 16)]
                    dv = dst_v[c, pl.ds(j * 16, 16)]
                    m = dv < filter_below
                    mi = m.astype(i32)
                    cnt = plsc.all_reduce_population_count(m)
                    pos = plsc.cumsum(mi) + (off_vec - 1)
                    idx = jnp.where(m, pos, trash)
                    row = jnp.right_shift(idx, 7)
                    col = jnp.bitwise_and(idx, CHUNK - 1)
                    plsc.store_scatter(src_v, [row, col], sv)
                    plsc.store_scatter(dst_v, [row, col], dv)
                    off_vec = off_vec + cnt
                return off_vec
            nedge = lax.fori_loop(
                0, nch, _scan_chunk, jnp.zeros((16,), i32))[0]

            for k in range(2 * CHUNK // 16):
                pp = nedge + 16 * k + lanes16
                prow = jnp.right_shift(pp, 7)
                pcol = jnp.bitwise_and(pp, CHUNK - 1)
                plsc.store_scatter(
                    src_v, [prow, pcol], 16 * k + lanes16)
                plsc.store_scatter(
                    dst_v, [prow, pcol],
                    filter_below + ((16 * k + lanes16) % CHUNK))
            nch_d = jnp.maximum(2 * ((nedge + 2 * CHUNK - 1) // (2 * CHUNK)),
                                2)
        else:
            nch_d = nch

        def _gidx(g):
            return src_v.at[g]

        def _didx(g):
            return dst_v.at[g]

        pltpu.async_copy(table_hbm.at[_gidx(0)], rows0, sem)

        def _swait():
            pltpu.make_async_copy(rows0, acc.at[_didx(0)], sem_s).wait()

        def _pair(h, carry):
            g0 = 2 * h
            g1 = g0 + 1
            pltpu.make_async_copy(table_hbm.at[_gidx(g0)], rows0, sem).wait()

            @pl.when(h > 0)
            def _drain_prev():
                _swait()

            pltpu.async_copy(table_hbm.at[_gidx(g1)], rows1, sem)
            pltpu.async_copy(rows0, acc.at[_didx(g0)], sem_s, add=True)
            pltpu.make_async_copy(table_hbm.at[_gidx(g1)], rows1, sem).wait()
            _swait()

            @pl.when(g1 + 1 < nch_d)
            def _prefetch():
                pltpu.async_copy(table_hbm.at[_gidx(g1 + 1)], rows0, sem)

            pltpu.async_copy(rows1, acc.at[_didx(g1)], sem_s, add=True)
            return carry
        lax.fori_loop(0, nch_d // 2, _pair, 0)
        _swait()
        plsc.subcore_barrier()

        off = 0
        while off < rpt:
            n = min(CHUNK, rpt - off)
            pltpu.sync_copy(acc.at[pl.ds(r0 + off, n)],
                            out_hbm.at[cid, pl.ds(r0 + off, n)])
            off += n

    return seg_kernel


_seg1 = _make_seg_sum(NCH1, T1_ROWS, ACC1_ROWS, filter_below=N2)
_seg2 = _make_seg_sum(NCH2, T2_ROWS, ACC2_ROWS)


def _tc1_body(p_ref, xd_ref, wl_ref, b_ref, wr_ref, o_ref):
    s = p_ref[0, :N2] + p_ref[1, :N2]
    cnt = s[:, D:D + 1]
    mean = s[:, :D] / jnp.maximum(cnt, 1.0)
    h = mean @ wl_ref[...] + b_ref[...] + xd_ref[...] @ wr_ref[...]
    h = jnp.maximum(h, 0.0)
    o_ref[...] = jnp.concatenate(
        [h, jnp.ones((N2, 1), jnp.float32), jnp.zeros((N2, DA - D - 1),
                                                      jnp.float32)], axis=1)


def _tc2_body(p_ref, h_ref, wl_ref, b_ref, wr_ref, o_ref):
    s = p_ref[0] + p_ref[1]
    mean = s[:, :D] / jnp.maximum(s[:, D:D + 1], 1.0)
    z = mean @ wl_ref[...] + b_ref[...] + h_ref[...][:, :D] @ wr_ref[...]
    m = jnp.max(z, axis=1, keepdims=True)
    lse = jnp.log(jnp.sum(jnp.exp(z - m), axis=1, keepdims=True)) + m
    o_ref[...] = z - lse


_tc1 = pl.pallas_call(
    _tc1_body, out_shape=jax.ShapeDtypeStruct((N2, DA), jnp.float32))
_tc2 = pl.pallas_call(
    _tc2_body, out_shape=jax.ShapeDtypeStruct((N2, D), jnp.float32))


def kernel(x, edge_index1, edge_index2, W1_l, b1_l, W1_r, W2_l, b2_l, W2_r):
    f32 = jnp.float32
    i32 = jnp.int32
    pad1 = E1P - E1
    pads = jnp.concatenate(
        [jnp.zeros((1, pad1), i32), jnp.full((1, pad1), N2, i32)], axis=0)
    e1 = jnp.concatenate(
        [edge_index1.astype(i32), pads], axis=1).reshape(2, -1, CHUNK)
    e2 = edge_index2.astype(i32).reshape(2, -1, CHUNK)

    xa = jnp.concatenate(
        [x[:N1], jnp.ones((N1, 1), f32), jnp.zeros((N1, DA - D - 1), f32)],
        axis=1)

    part1 = _seg1(xa, e1)
    ha = _tc1(part1, x[:N2], W1_l, b1_l.reshape(1, D), W1_r)

    part2 = _seg2(ha, e2)
    out = _tc2(part2, ha, W2_l, b2_l.reshape(1, D), W2_r)
    return out
